# baseline (speedup 1.0000x reference)
import functools

import jax
import jax.numpy as jnp
from jax import lax
from jax.experimental import pallas as pl
from jax.experimental.pallas import tpu as pltpu
from jax.experimental.pallas import tpu_sc as plsc

NC = 2
NS = 16
NW = NC * NS
LANES = 16
K = 128

F32 = jnp.float32
I32 = jnp.int32
LOMASK = 0xFFFF


def _mesh():
    return plsc.VectorSubcoreMesh(core_axis_name="c", subcore_axis_name="s")


def _unpack(pv):
    return pv & LOMASK, lax.shift_right_logical(pv, 16)


def _sc_degrees(P, EPW):
    @functools.partial(
        pl.kernel,
        out_type=jax.ShapeDtypeStruct((NW, P), F32),
        mesh=_mesh(),
        compiler_params=pltpu.CompilerParams(needs_layout_passes=False),
        scratch_types=[
            pltpu.VMEM((EPW,), I32),
            pltpu.VMEM((P,), F32),
        ],
    )
    def k(pk_hbm, out_hbm, pk_v, acc_v):
        c = lax.axis_index("c")
        s = lax.axis_index("s")
        w = c * NS + s

        def zero(i, _):
            acc_v[pl.ds(i * LANES, LANES)] = jnp.zeros((LANES,), F32)
            return 0

        lax.fori_loop(0, P // LANES, zero, 0)
        pltpu.sync_copy(pk_hbm.at[w], pk_v)
        ones16 = jnp.ones((LANES,), F32)

        def body(j, _):
            _, didx = _unpack(pk_v[pl.ds(j * LANES, LANES)])
            plsc.addupdate_scatter(acc_v, [didx], ones16)
            return 0

        lax.fori_loop(0, EPW // LANES, body, 0)
        pltpu.sync_copy(acc_v, out_hbm.at[w])

    return k


def _sc_rows(P, NCHUNK):
    STRIPE = P // NS

    @functools.partial(
        pl.kernel,
        out_type=jax.ShapeDtypeStruct((NC, P, 128), F32),
        mesh=_mesh(),
        compiler_params=pltpu.CompilerParams(needs_layout_passes=False),
        scratch_types=[
            pltpu.VMEM((K,), I32),
            pltpu.VMEM((K,), I32),
            pltpu.VMEM((K,), I32),
            pltpu.VMEM((K,), I32),
            pltpu.VMEM((K,), I32),
            pltpu.VMEM((K,), I32),
            pltpu.VMEM((K, 128), F32),
            pltpu.VMEM((K, 128), F32),
            pltpu.VMEM_SHARED((P, 128), F32),
            pltpu.SemaphoreType.DMA,
            pltpu.SemaphoreType.DMA,
            pltpu.SemaphoreType.DMA,
            pltpu.SemaphoreType.DMA,
        ],
    )
    def k(g_hbm, pk_hbm, out_hbm, pk_a, pk_b, sid_a, did_a, sid_b, did_b,
          rows_a, rows_b, acc_sh, sem_pa, sem_pb, sem_a, sem_b):
        c = lax.axis_index("c")
        s = lax.axis_index("s")
        w = c * NS + s

        def unpack(pk_v, sid_v, did_v):
            for j in range(K // LANES):
                sl = pl.ds(j * LANES, LANES)
                sidx, didx = _unpack(pk_v[sl])
                sid_v[sl] = sidx
                did_v[sl] = didx

        zero16 = jnp.zeros((LANES,), F32)

        def zrow(r, _):
            for j in range(128 // LANES):
                rows_a[r, pl.ds(j * LANES, LANES)] = zero16
            return 0

        lax.fori_loop(0, K, zrow, 0)
        for t in range(STRIPE // K):
            pltpu.sync_copy(rows_a, acc_sh.at[pl.ds(s * STRIPE + t * K, K)])
        plsc.subcore_barrier()

        pltpu.sync_copy(pk_hbm.at[w, 0], pk_a)
        unpack(pk_a, sid_a, did_a)
        pltpu.async_copy(pk_hbm.at[w, 1], pk_b, sem_pb)
        pltpu.async_copy(g_hbm.at[sid_a], rows_a, sem_a)

        def chunk2(k2, _):
            i0 = 2 * k2
            pltpu.make_async_copy(pk_hbm.at[w, i0 + 1], pk_b, sem_pb).wait()
            unpack(pk_b, sid_b, did_b)
            cp_b = pltpu.async_copy(g_hbm.at[sid_b], rows_b, sem_b)

            @pl.when(i0 + 2 < NCHUNK)
            def _():
                pltpu.async_copy(pk_hbm.at[w, i0 + 2], pk_a, sem_pa)

            pltpu.make_async_copy(g_hbm.at[sid_a], rows_a, sem_a).wait()
            pltpu.sync_copy(rows_a, acc_sh.at[did_a], add=True)

            @pl.when(i0 + 2 < NCHUNK)
            def _():
                pltpu.make_async_copy(pk_hbm.at[w, i0 + 2], pk_a, sem_pa).wait()
                unpack(pk_a, sid_a, did_a)
                pltpu.async_copy(g_hbm.at[sid_a], rows_a, sem_a)

            cp_b.wait()
            pltpu.sync_copy(rows_b, acc_sh.at[did_b], add=True)

            @pl.when(i0 + 3 < NCHUNK)
            def _():
                pltpu.async_copy(pk_hbm.at[w, i0 + 3], pk_b, sem_pb)

            return 0

        lax.fori_loop(0, NCHUNK // 2, chunk2, 0)
        plsc.subcore_barrier()
        for t in range(STRIPE // K):
            sl = pl.ds(s * STRIPE + t * K, K)
            pltpu.sync_copy(acc_sh.at[sl], rows_a)
            pltpu.sync_copy(rows_a, out_hbm.at[c, sl])

    return k


def _sc_scalars(P, EPW):
    @functools.partial(
        pl.kernel,
        out_type=jax.ShapeDtypeStruct((NW, P), F32),
        mesh=_mesh(),
        compiler_params=pltpu.CompilerParams(needs_layout_passes=False),
        scratch_types=[
            pltpu.VMEM((EPW,), I32),
            pltpu.VMEM((P,), F32),
            pltpu.VMEM((P,), F32),
        ],
    )
    def k(zs_hbm, pk_hbm, out_hbm, pk_v, zs_v, acc_v):
        c = lax.axis_index("c")
        s = lax.axis_index("s")
        w = c * NS + s
        pltpu.sync_copy(zs_hbm, zs_v)
        pltpu.sync_copy(pk_hbm.at[w], pk_v)

        def zero(i, _):
            acc_v[pl.ds(i * LANES, LANES)] = jnp.zeros((LANES,), F32)
            return 0

        lax.fori_loop(0, P // LANES, zero, 0)

        def body(j, _):
            sidx, didx = _unpack(pk_v[pl.ds(j * LANES, LANES)])
            vals = plsc.load_gather(zs_v, [sidx])
            plsc.addupdate_scatter(acc_v, [didx], vals)
            return 0

        lax.fori_loop(0, EPW // LANES, body, 0)
        pltpu.sync_copy(acc_v, out_hbm.at[w])

    return k


def _tc_g(degT, x_pad, W1, P, BR):
    grid = (P // BR,)

    def body(deg_ref, x_ref, w1_ref, g_ref):
        deg = jnp.sum(deg_ref[...], axis=1, keepdims=True) + 1.0
        dis = lax.rsqrt(deg)
        h = jnp.dot(x_ref[...], w1_ref[...], preferred_element_type=F32)
        g_ref[...] = dis * h

    return pl.pallas_call(
        body,
        grid=grid,
        in_specs=[
            pl.BlockSpec((BR, NW), lambda i: (i, 0)),
            pl.BlockSpec((BR, 128), lambda i: (i, 0)),
            pl.BlockSpec((128, 128), lambda i: (0, 0)),
        ],
        out_specs=pl.BlockSpec((BR, 128), lambda i: (i, 0)),
        out_shape=jax.ShapeDtypeStruct((P, 128), F32),
    )(degT, x_pad, W1)


def _tc_zs(acc_part, g, degT, b1r, w2r, P, BR):
    grid = (P // BR,)

    def body(acc_ref, g_ref, deg_ref, b1_ref, w2_ref, zs_ref):
        acc = acc_ref[0] + acc_ref[1]
        deg = jnp.sum(deg_ref[...], axis=1, keepdims=True) + 1.0
        dis = lax.rsqrt(deg)
        h1 = jnp.maximum(dis * (acc + g_ref[...]) + b1_ref[...], 0.0)
        z = jnp.sum(h1 * w2_ref[...], axis=1, keepdims=True)
        zs_ref[...] = dis * z

    return pl.pallas_call(
        body,
        grid=grid,
        in_specs=[
            pl.BlockSpec((NC, BR, 128), lambda i: (0, i, 0)),
            pl.BlockSpec((BR, 128), lambda i: (i, 0)),
            pl.BlockSpec((BR, NW), lambda i: (i, 0)),
            pl.BlockSpec((1, 128), lambda i: (0, 0)),
            pl.BlockSpec((1, 128), lambda i: (0, 0)),
        ],
        out_specs=pl.BlockSpec((BR, 1), lambda i: (i, 0)),
        out_shape=jax.ShapeDtypeStruct((P, 1), F32),
    )(acc_part, g, degT, b1r, w2r)


def _tc_out(sacc2, zs2, deg2, b2r, P):
    R = P // 128

    def body(sacc_ref, zs_ref, deg_ref, b2_ref, out_ref):
        sacc = jnp.sum(sacc_ref[...], axis=0)
        deg = jnp.sum(deg_ref[...], axis=0) + 1.0
        dis = lax.rsqrt(deg)
        out_ref[...] = dis * (sacc + zs_ref[...]) + b2_ref[0, 0]

    return pl.pallas_call(
        body,
        out_shape=jax.ShapeDtypeStruct((R, 128), F32),
    )(sacc2, zs2, deg2, b2r)


def kernel(x, edge_index, W1, b1, W2, b2):
    N, D = x.shape
    H = W1.shape[1]
    E = edge_index.shape[1]
    src = edge_index[0].astype(I32)
    dst = edge_index[1].astype(I32)

    P = -(-(N + 1) // (NS * K)) * (NS * K)
    EPW = -(-E // (NW * 2 * K)) * 2 * K
    EPAD = EPW * NW
    NCHUNK = EPW // K
    BR = 512

    x_pad = jnp.zeros((P, D), F32).at[:N].set(x)
    packed = jnp.left_shift(dst, 16) | src
    pk = jnp.full((EPAD,), (N << 16) | N, I32).at[:E].set(packed)
    pk2 = pk.reshape(NW, EPW)
    pk3 = pk.reshape(NW, NCHUNK, K)

    deg_part = _sc_degrees(P, EPW)(pk2)
    degT = deg_part.T

    g = _tc_g(degT, x_pad, W1, P, BR)
    acc_part = _sc_rows(P, NCHUNK)(g, pk3)

    b1r = b1.reshape(1, H)
    w2r = W2.reshape(1, H)
    zs = _tc_zs(acc_part, g, degT, b1r, w2r, P, BR)

    sacc_part = _sc_scalars(P, EPW)(zs.reshape(P), pk2)

    out2 = _tc_out(
        sacc_part.reshape(NW, P // 128, 128),
        zs.reshape(P // 128, 128),
        deg_part.reshape(NW, P // 128, 128),
        b2.reshape(1, 1),
        P,
    )
    return out2.reshape(-1)[:N]


# speedup vs baseline: 2.5087x; 2.5087x over previous
import functools

import jax
import jax.numpy as jnp
from jax import lax
from jax.experimental import pallas as pl
from jax.experimental.pallas import tpu as pltpu
from jax.experimental.pallas import tpu_sc as plsc

NC = 2
NS = 16
NW = NC * NS
LANES = 16
K = 128

F32 = jnp.float32
I32 = jnp.int32
LOMASK = 0xFFFF


def _mesh():
    return plsc.VectorSubcoreMesh(core_axis_name="c", subcore_axis_name="s")


def _unpack(pv):
    return pv & LOMASK, lax.shift_right_logical(pv, 16)


def _sc_degrees(P, EPW):
    @functools.partial(
        pl.kernel,
        out_type=jax.ShapeDtypeStruct((NW, P), F32),
        mesh=_mesh(),
        compiler_params=pltpu.CompilerParams(needs_layout_passes=False),
        scratch_types=[
            pltpu.VMEM((EPW,), I32),
            pltpu.VMEM((P,), F32),
        ],
    )
    def k(pk_hbm, out_hbm, pk_v, acc_v):
        c = lax.axis_index("c")
        s = lax.axis_index("s")
        w = c * NS + s

        def zero(i, _):
            acc_v[pl.ds(i * LANES, LANES)] = jnp.zeros((LANES,), F32)
            return 0

        lax.fori_loop(0, P // LANES, zero, 0)
        pltpu.sync_copy(pk_hbm.at[w], pk_v)
        ones16 = jnp.ones((LANES,), F32)

        def body(j, _):
            _, didx = _unpack(pk_v[pl.ds(j * LANES, LANES)])
            plsc.addupdate_scatter(acc_v, [didx], ones16)
            return 0

        lax.fori_loop(0, EPW // LANES, body, 0)
        pltpu.sync_copy(acc_v, out_hbm.at[w])

    return k


def _sc_rows(P, NCHUNK):
    STRIPE = P // NS

    @functools.partial(
        pl.kernel,
        out_type=jax.ShapeDtypeStruct((NC, P, 128), F32),
        mesh=_mesh(),
        compiler_params=pltpu.CompilerParams(needs_layout_passes=False),
        scratch_types=[
            pltpu.VMEM((K,), I32),
            pltpu.VMEM((K,), I32),
            pltpu.VMEM((K,), I32),
            pltpu.VMEM((K,), I32),
            pltpu.VMEM((K,), I32),
            pltpu.VMEM((K,), I32),
            pltpu.VMEM((K, 128), F32),
            pltpu.VMEM((K, 128), F32),
            pltpu.VMEM_SHARED((P, 128), F32),
            pltpu.SemaphoreType.DMA,
            pltpu.SemaphoreType.DMA,
            pltpu.SemaphoreType.DMA,
            pltpu.SemaphoreType.DMA,
        ],
    )
    def k(g_hbm, pk_hbm, out_hbm, pk_a, pk_b, sid_a, did_a, sid_b, did_b,
          rows_a, rows_b, acc_sh, sem_pa, sem_pb, sem_a, sem_b):
        c = lax.axis_index("c")
        s = lax.axis_index("s")
        w = c * NS + s

        def unpack(pk_v, sid_v, did_v):
            for j in range(K // LANES):
                sl = pl.ds(j * LANES, LANES)
                sidx, didx = _unpack(pk_v[sl])
                sid_v[sl] = sidx
                did_v[sl] = didx

        zero16 = jnp.zeros((LANES,), F32)

        def zrow(r, _):
            for j in range(128 // LANES):
                rows_a[r, pl.ds(j * LANES, LANES)] = zero16
            return 0

        lax.fori_loop(0, K, zrow, 0)
        for t in range(STRIPE // K):
            pltpu.sync_copy(rows_a, acc_sh.at[pl.ds(s * STRIPE + t * K, K)])
        plsc.subcore_barrier()

        pltpu.sync_copy(pk_hbm.at[w, 0], pk_a)
        unpack(pk_a, sid_a, did_a)
        pltpu.async_copy(pk_hbm.at[w, 1], pk_b, sem_pb)
        pltpu.async_copy(g_hbm.at[sid_a], rows_a, sem_a)

        def chunk2(k2, _):
            i0 = 2 * k2
            pltpu.make_async_copy(pk_hbm.at[w, i0 + 1], pk_b, sem_pb).wait()
            unpack(pk_b, sid_b, did_b)
            cp_b = pltpu.async_copy(g_hbm.at[sid_b], rows_b, sem_b)

            @pl.when(i0 + 2 < NCHUNK)
            def _():
                pltpu.async_copy(pk_hbm.at[w, i0 + 2], pk_a, sem_pa)

            pltpu.make_async_copy(g_hbm.at[sid_a], rows_a, sem_a).wait()
            pltpu.sync_copy(rows_a, acc_sh.at[did_a], add=True)

            @pl.when(i0 + 2 < NCHUNK)
            def _():
                pltpu.make_async_copy(pk_hbm.at[w, i0 + 2], pk_a, sem_pa).wait()
                unpack(pk_a, sid_a, did_a)
                pltpu.async_copy(g_hbm.at[sid_a], rows_a, sem_a)

            cp_b.wait()
            pltpu.sync_copy(rows_b, acc_sh.at[did_b], add=True)

            @pl.when(i0 + 3 < NCHUNK)
            def _():
                pltpu.async_copy(pk_hbm.at[w, i0 + 3], pk_b, sem_pb)

            return 0

        lax.fori_loop(0, NCHUNK // 2, chunk2, 0)
        plsc.subcore_barrier()
        for t in range(STRIPE // K):
            sl = pl.ds(s * STRIPE + t * K, K)
            pltpu.sync_copy(acc_sh.at[sl], rows_a)
            pltpu.sync_copy(rows_a, out_hbm.at[c, sl])

    return k


def _sc_scalars(P, EPW):
    @functools.partial(
        pl.kernel,
        out_type=jax.ShapeDtypeStruct((NW, P), F32),
        mesh=_mesh(),
        compiler_params=pltpu.CompilerParams(needs_layout_passes=False),
        scratch_types=[
            pltpu.VMEM((EPW,), I32),
            pltpu.VMEM((P,), F32),
            pltpu.VMEM((P,), F32),
        ],
    )
    def k(zs_hbm, pk_hbm, out_hbm, pk_v, zs_v, acc_v):
        c = lax.axis_index("c")
        s = lax.axis_index("s")
        w = c * NS + s
        pltpu.sync_copy(zs_hbm, zs_v)
        pltpu.sync_copy(pk_hbm.at[w], pk_v)

        def zero(i, _):
            acc_v[pl.ds(i * LANES, LANES)] = jnp.zeros((LANES,), F32)
            return 0

        lax.fori_loop(0, P // LANES, zero, 0)

        def body(j, _):
            sidx, didx = _unpack(pk_v[pl.ds(j * LANES, LANES)])
            vals = plsc.load_gather(zs_v, [sidx])
            plsc.addupdate_scatter(acc_v, [didx], vals)
            return 0

        lax.fori_loop(0, EPW // LANES, body, 0)
        pltpu.sync_copy(acc_v, out_hbm.at[w])

    return k


def _tc_g(degT, x_pad, W1, P, BR):
    grid = (P // BR,)

    def body(deg_ref, x_ref, w1_ref, g_ref):
        deg = jnp.sum(deg_ref[...], axis=1, keepdims=True) + 1.0
        dis = lax.rsqrt(deg)
        h = jnp.dot(x_ref[...], w1_ref[...], preferred_element_type=F32)
        g_ref[...] = dis * h

    return pl.pallas_call(
        body,
        grid=grid,
        in_specs=[
            pl.BlockSpec((BR, NW), lambda i: (i, 0)),
            pl.BlockSpec((BR, 128), lambda i: (i, 0)),
            pl.BlockSpec((128, 128), lambda i: (0, 0)),
        ],
        out_specs=pl.BlockSpec((BR, 128), lambda i: (i, 0)),
        out_shape=jax.ShapeDtypeStruct((P, 128), F32),
    )(degT, x_pad, W1)


def _tc_zs(acc_part, g, degT, b1r, w2r, P, BR):
    grid = (P // BR,)

    def body(acc_ref, g_ref, deg_ref, b1_ref, w2_ref, zs_ref):
        acc = acc_ref[0] + acc_ref[1]
        deg = jnp.sum(deg_ref[...], axis=1, keepdims=True) + 1.0
        dis = lax.rsqrt(deg)
        h1 = jnp.maximum(dis * (acc + g_ref[...]) + b1_ref[...], 0.0)
        z = jnp.sum(h1 * w2_ref[...], axis=1, keepdims=True)
        zs_ref[...] = dis * z

    return pl.pallas_call(
        body,
        grid=grid,
        in_specs=[
            pl.BlockSpec((NC, BR, 128), lambda i: (0, i, 0)),
            pl.BlockSpec((BR, 128), lambda i: (i, 0)),
            pl.BlockSpec((BR, NW), lambda i: (i, 0)),
            pl.BlockSpec((1, 128), lambda i: (0, 0)),
            pl.BlockSpec((1, 128), lambda i: (0, 0)),
        ],
        out_specs=pl.BlockSpec((BR, 1), lambda i: (i, 0)),
        out_shape=jax.ShapeDtypeStruct((P, 1), F32),
    )(acc_part, g, degT, b1r, w2r)


def _tc_out(sacc2, zs2, deg2, b2r, P):
    R = P // 128

    def body(sacc_ref, zs_ref, deg_ref, b2_ref, out_ref):
        sacc = jnp.sum(sacc_ref[...], axis=0)
        deg = jnp.sum(deg_ref[...], axis=0) + 1.0
        dis = lax.rsqrt(deg)
        out_ref[...] = dis * (sacc + zs_ref[...]) + b2_ref[0, 0]

    return pl.pallas_call(
        body,
        out_shape=jax.ShapeDtypeStruct((R, 128), F32),
    )(sacc2, zs2, deg2, b2r)


def kernel(x, edge_index, W1, b1, W2, b2):
    N, D = x.shape
    H = W1.shape[1]
    E = edge_index.shape[1]
    src = edge_index[0].astype(I32)
    dst = edge_index[1].astype(I32)

    P = -(-(N + 1) // (NS * K)) * (NS * K)
    EPW = -(-E // (NW * 2 * K)) * 2 * K
    EPAD = EPW * NW
    NCHUNK = EPW // K
    BR = 512

    x_pad = jnp.zeros((P, D), F32).at[:N].set(x)
    packed = jnp.left_shift(dst, 16) | src
    dum = N + jnp.arange(EPAD - E, dtype=I32) % (P - N)
    pk = jnp.concatenate([packed, jnp.left_shift(dum, 16) | dum])
    pk2 = pk.reshape(NW, EPW)
    pk3 = pk.reshape(NW, NCHUNK, K)

    deg_part = _sc_degrees(P, EPW)(pk2)
    degT = deg_part.T

    g = _tc_g(degT, x_pad, W1, P, BR)
    acc_part = _sc_rows(P, NCHUNK)(g, pk3)

    b1r = b1.reshape(1, H)
    w2r = W2.reshape(1, H)
    zs = _tc_zs(acc_part, g, degT, b1r, w2r, P, BR)

    sacc_part = _sc_scalars(P, EPW)(zs.reshape(P), pk2)

    out2 = _tc_out(
        sacc_part.reshape(NW, P // 128, 128),
        zs.reshape(P // 128, 128),
        deg_part.reshape(NW, P // 128, 128),
        b2.reshape(1, 1),
        P,
    )
    return out2.reshape(-1)[:N]
